# baseline (speedup 1.0000x reference)
import functools

import jax
import jax.numpy as jnp
import numpy as np
from jax import lax
from jax.experimental import pallas as pl
from jax.experimental.pallas import tpu as pltpu
from jax.experimental.pallas import tpu_sc as plsc

N_NODES = 50000
N_EDGES = 800000
BE = 8000
BN = 2000

NC = 2
NW = 32
PER_W = N_EDGES // NW
GC = 1000
ITERS = PER_W // GC

BF = jnp.bfloat16
F32 = jnp.float32


@functools.cache
def _build_sc_gather():
    mesh = plsc.VectorSubcoreMesh(core_axis_name="c", subcore_axis_name="s")

    @functools.partial(
        pl.kernel, mesh=mesh,
        compiler_params=pltpu.CompilerParams(use_tc_tiling_on_sc=False),
        out_type=jax.ShapeDtypeStruct((2 * N_EDGES, 16), F32),
        scratch_types=[pltpu.VMEM((2 * GC,), jnp.int32),
                       pltpu.VMEM((2 * GC, 16), F32),
                       pltpu.VMEM_SHARED((N_NODES, 16), F32),
                       pltpu.SemaphoreType.DMA],
    )
    def _sc_gather_kernel(h_hbm, idx2_hbm, out_hbm, iv, rbuf, hsh, sem):
        sid = lax.axis_index("s")
        wid = sid * NC + lax.axis_index("c")
        base = wid * PER_W

        @pl.when(sid == 0)
        def _():
            pltpu.sync_copy(h_hbm, hsh)

        plsc.subcore_barrier()

        def body(j, carry):
            off2 = 2 * (base + j * GC)
            pltpu.sync_copy(idx2_hbm.at[pl.ds(off2, 2 * GC)], iv)
            pltpu.async_copy(hsh.at[iv], rbuf, sem).wait()
            pltpu.sync_copy(rbuf, out_hbm.at[pl.ds(off2, 2 * GC)])
            return carry

        lax.fori_loop(0, ITERS, body, 0)

    return _sc_gather_kernel


@functools.cache
def _build_sc_scatter():
    mesh = plsc.VectorSubcoreMesh(core_axis_name="c", subcore_axis_name="s")

    @functools.partial(
        pl.kernel, mesh=mesh,
        compiler_params=pltpu.CompilerParams(use_tc_tiling_on_sc=False),
        out_type=jax.ShapeDtypeStruct((NC, N_NODES, 16), F32),
        scratch_types=[pltpu.VMEM((GC, 16), F32),
                       pltpu.VMEM((GC,), jnp.int32),
                       pltpu.VMEM_SHARED((N_NODES, 16), F32)],
    )
    def _sc_scatter_kernel(m_hbm, dst_hbm, zeros_hbm, out_hbm, mv, dv, acc):
        cid = lax.axis_index("c")
        sid = lax.axis_index("s")
        wid = sid * NC + cid

        @pl.when(sid == 0)
        def _():
            pltpu.sync_copy(zeros_hbm, acc)

        plsc.subcore_barrier()
        base = wid * PER_W

        def body(j, carry):
            off = base + j * GC
            pltpu.sync_copy(m_hbm.at[pl.ds(off, GC)], mv)
            pltpu.sync_copy(dst_hbm.at[pl.ds(off, GC)], dv)
            pltpu.sync_copy(mv, acc.at[dv], add=True)
            return carry

        lax.fori_loop(0, ITERS, body, 0)
        plsc.subcore_barrier()
        rows = N_NODES // 16
        pltpu.sync_copy(acc.at[pl.ds(sid * rows, rows)],
                        out_hbm.at[cid, pl.ds(sid * rows, rows)])

    return _sc_scatter_kernel


@functools.cache
def _build_sc_counts():
    mesh = plsc.VectorSubcoreMesh(core_axis_name="c", subcore_axis_name="s")

    @functools.partial(
        pl.kernel, mesh=mesh,
        compiler_params=pltpu.CompilerParams(use_tc_tiling_on_sc=False),
        out_type=jax.ShapeDtypeStruct((NC, N_NODES, 16), F32),
        scratch_types=[pltpu.VMEM((GC, 16), F32),
                       pltpu.VMEM((GC,), jnp.int32),
                       pltpu.VMEM_SHARED((N_NODES, 16), F32)],
    )
    def _sc_counts_kernel(dst_hbm, ones_hbm, zeros_hbm, out_hbm, ov, dv, acc):
        cid = lax.axis_index("c")
        sid = lax.axis_index("s")
        wid = sid * NC + cid

        pltpu.sync_copy(ones_hbm, ov)

        @pl.when(sid == 0)
        def _():
            pltpu.sync_copy(zeros_hbm, acc)

        plsc.subcore_barrier()
        base = wid * PER_W

        def body(j, carry):
            off = base + j * GC
            pltpu.sync_copy(dst_hbm.at[pl.ds(off, GC)], dv)
            pltpu.sync_copy(ov, acc.at[dv], add=True)
            return carry

        lax.fori_loop(0, ITERS, body, 0)
        plsc.subcore_barrier()
        rows = N_NODES // 16
        pltpu.sync_copy(acc.at[pl.ds(sid * rows, rows)],
                        out_hbm.at[cid, pl.ds(sid * rows, rows)])

    return _sc_counts_kernel


def _sc_gather(h_flat, idx2):
    return _build_sc_gather()(h_flat, idx2)


def _sc_scatter(m_flat, dst, zeros_n16):
    return _build_sc_scatter()(m_flat, dst, zeros_n16)


def _sc_counts(dst, ones_gc16, zeros_n16):
    return _build_sc_counts()(dst, ones_gc16, zeros_n16)


def _dot_f32(a, b):
    return jax.lax.dot_general(a, b, (((1,), (0,)), ((), ())),
                               preferred_element_type=F32)


def _enc_body(x_ref, w_ref, b_ref, o_ref):
    o_ref[...] = _dot_f32(x_ref[...], w_ref[...]) + b_ref[...]


def _msg_body(xb_ref, eb_ref, w0, b0, w1, b1, w2, b2, w3, b3, o_ref):
    xin = jnp.concatenate([xb_ref[...], eb_ref[...]], axis=1).astype(BF)
    m = jnp.maximum(_dot_f32(xin, w0[...]) + b0[...], 0.0)
    m = jnp.maximum(_dot_f32(m.astype(BF), w1[...]) + b1[...], 0.0)
    m = jnp.maximum(_dot_f32(m.astype(BF), w2[...]) + b2[...], 0.0)
    m = _dot_f32(m.astype(BF), w3[...]) + b3[...]
    o_ref[...] = jnp.concatenate([m[:BE // 8], m[BE // 8:]], axis=1)


def _upd_body(h_ref, s0_ref, s1_ref, c0_ref, c1_ref, w0, b0, w1, b1, o_ref):
    cnt = jnp.maximum(c0_ref[...] + c1_ref[...], 1.0)
    aggr = (s0_ref[...] + s1_ref[...]) / cnt
    u_in = jnp.concatenate([h_ref[...], aggr], axis=1)
    u = jnp.maximum(_dot_f32(u_in, w0[...]) + b0[...], 0.0)
    o_ref[...] = _dot_f32(u, w1[...]) + b1[...]


def _final_body(h_ref, w0, b0, w1, b1, w2, b2, o_ref):
    p = jnp.maximum(_dot_f32(h_ref[...], w0[...]) + b0[...], 0.0)
    p = jnp.maximum(_dot_f32(p, w1[...]) + b1[...], 0.0)
    p = _dot_f32(p, w2[...]) + b2[...]
    part = jnp.sum(p, axis=0, keepdims=True)

    @pl.when(pl.program_id(0) == 0)
    def _():
        o_ref[...] = jnp.zeros_like(o_ref)

    o_ref[...] += part


def _full_spec(shape):
    return pl.BlockSpec(shape, lambda i: tuple(0 for _ in shape))


NR = N_NODES // 8


def _encoder(x8, enc_rep, enc_b_rep):
    return pl.pallas_call(
        _enc_body,
        grid=(1,),
        in_specs=[
            pl.BlockSpec((NR, 128), lambda i: (0, 0)),
            _full_spec((128, 128)),
            _full_spec((1, 128)),
        ],
        out_specs=pl.BlockSpec((NR, 128), lambda i: (0, 0)),
        out_shape=jax.ShapeDtypeStruct((NR, 128), F32),
    )(x8, enc_rep, enc_b_rep)


def _msg_mlp(xb, eb, w0, b0, w1, b1, w2, b2, w3, b3):
    return pl.pallas_call(
        _msg_body,
        grid=(N_EDGES // BE,),
        in_specs=[
            pl.BlockSpec((BE // 4, 128), lambda i: (i, 0)),
            pl.BlockSpec((BE // 4, 128), lambda i: (i, 0)),
            _full_spec((256, 280)),
            _full_spec((1, 280)),
            _full_spec((280, 560)),
            _full_spec((1, 560)),
            _full_spec((560, 80)),
            _full_spec((1, 80)),
            _full_spec((80, 64)),
            _full_spec((1, 64)),
        ],
        out_specs=pl.BlockSpec((BE // 8, 128), lambda i: (i, 0)),
        out_shape=jax.ShapeDtypeStruct((N_EDGES // 8, 128), F32),
    )(xb, eb, w0, b0, w1, b1, w2, b2, w3, b3)


def _update(h8, s0, s1, c0, c1, w0, b0, w1, b1):
    return pl.pallas_call(
        _upd_body,
        grid=(1,),
        in_specs=[
            pl.BlockSpec((NR, 128), lambda i: (0, 0)),
            pl.BlockSpec((NR, 128), lambda i: (0, 0)),
            pl.BlockSpec((NR, 128), lambda i: (0, 0)),
            pl.BlockSpec((NR, 128), lambda i: (0, 0)),
            pl.BlockSpec((NR, 128), lambda i: (0, 0)),
            _full_spec((256, 560)),
            _full_spec((1, 560)),
            _full_spec((560, 128)),
            _full_spec((1, 128)),
        ],
        out_specs=pl.BlockSpec((NR, 128), lambda i: (0, 0)),
        out_shape=jax.ShapeDtypeStruct((NR, 128), F32),
    )(h8, s0, s1, c0, c1, w0, b0, w1, b1)


def _final(h8, w0, b0, w1, b1, w2, b2):
    out = pl.pallas_call(
        _final_body,
        grid=(1,),
        in_specs=[
            pl.BlockSpec((NR, 128), lambda i: (0, 0)),
            _full_spec((128, 512)),
            _full_spec((1, 512)),
            _full_spec((512, 256)),
            _full_spec((1, 256)),
            _full_spec((256, 24)),
            _full_spec((1, 24)),
        ],
        out_specs=_full_spec((1, 24)),
        out_shape=jax.ShapeDtypeStruct((1, 24), F32),
    )(h8, w0, b0, w1, b1, w2, b2)
    return out.reshape(8, 3).sum(axis=0, keepdims=True) / N_NODES


def _blockdiag(w, k, in_blk, out_blk):
    out = jnp.zeros((k * in_blk, k * out_blk), w.dtype)
    for i in range(k):
        out = out.at[i * in_blk:(i + 1) * in_blk,
                     i * out_blk:(i + 1) * out_blk].set(w)
    return out


def _tile_bias(b, k):
    return jnp.tile(b.reshape(1, -1), (1, k))


def kernel(x, edge_index, edge_attr, enc_W, enc_b,
           mW0, mb0, mW1, mb1, mW2, mb2, mW3, mb3,
           uW0, ub0, uW1, ub1,
           fW0, fb0, fW1, fb1, fW2, fb2):
    src = edge_index[0]
    dst = edge_index[1]
    idx2 = jnp.stack([dst, src], axis=1).reshape(2 * N_EDGES)
    ids = jnp.arange(N_EDGES).reshape(N_EDGES // BE, 2, BE // 8, 4)
    perm = jnp.transpose(ids, (0, 2, 1, 3)).reshape(-1)
    dst_perm = jnp.take(dst, perm)
    zeros_n16 = jnp.zeros((N_NODES, 16), F32)
    ones_gc16 = jnp.ones((GC, 16), F32)

    ea_pad = jnp.pad(edge_attr, ((0, 0), (0, 29)))
    eb = ea_pad.reshape(N_EDGES // 4, 128)

    x8 = x.reshape(N_NODES // 8, 128)
    enc_rep = _blockdiag(enc_W.T, 8, 16, 16)
    h8 = _encoder(x8, enc_rep, _tile_bias(enc_b, 8))

    cnt = _sc_counts(dst, ones_gc16, zeros_n16)
    c0 = cnt[0].reshape(N_NODES // 8, 128)
    c1 = cnt[1].reshape(N_NODES // 8, 128)

    for l in range(3):
        w0t = mW0[l].T
        w0x = _blockdiag(w0t[0:32], 4, 32, 70)
        w0e = _blockdiag(jnp.pad(w0t[32:35], ((0, 29), (0, 0))), 4, 32, 70)
        w0full = jnp.concatenate([w0x, w0e], axis=0).astype(BF)

        gath = _sc_gather(h8.reshape(N_NODES, 16), idx2)
        xb = gath.reshape(N_EDGES // 4, 128)

        m8 = _msg_mlp(xb, eb, w0full, _tile_bias(mb0[l], 4),
                      _blockdiag(mW1[l].T, 4, 70, 140).astype(BF),
                      _tile_bias(mb1[l], 4),
                      _blockdiag(mW2[l].T, 4, 140, 20).astype(BF),
                      _tile_bias(mb2[l], 4),
                      _blockdiag(mW3[l].T, 4, 20, 16).astype(BF),
                      _tile_bias(mb3[l], 4))

        s = _sc_scatter(m8.reshape(N_EDGES, 16), dst_perm, zeros_n16)
        u0t = uW0[l].T
        uw0 = jnp.concatenate([_blockdiag(u0t[0:16], 8, 16, 70),
                               _blockdiag(u0t[16:32], 8, 16, 70)], axis=0)
        h8 = _update(h8,
                     s[0].reshape(N_NODES // 8, 128),
                     s[1].reshape(N_NODES // 8, 128),
                     c0, c1,
                     uw0, _tile_bias(ub0[l], 8),
                     _blockdiag(uW1[l].T, 8, 70, 16), _tile_bias(ub1[l], 8))

    return _final(h8,
                  _blockdiag(fW0.T, 8, 16, 64), _tile_bias(fb0, 8),
                  _blockdiag(fW1.T, 8, 64, 32), _tile_bias(fb1, 8),
                  _blockdiag(fW2.T, 8, 32, 3), _tile_bias(fb2, 8))


# speedup vs baseline: 1.3187x; 1.3187x over previous
import functools

import jax
import jax.numpy as jnp
from jax import lax
from jax.experimental import pallas as pl
from jax.experimental.pallas import tpu as pltpu
from jax.experimental.pallas import tpu_sc as plsc

N_NODES = 50000
N_EDGES = 800000
BE = 8000
BN = 2000

NC = 2
NW = 32
PER_W = N_EDGES // NW
GC = 1000
ITERS = PER_W // GC
GS = 1000
ITERS_S = PER_W // GS

BF = jnp.bfloat16
F32 = jnp.float32


@functools.cache
def _build_sc_gather():
    mesh = plsc.VectorSubcoreMesh(core_axis_name="c", subcore_axis_name="s")

    @functools.partial(
        pl.kernel, mesh=mesh,
        compiler_params=pltpu.CompilerParams(use_tc_tiling_on_sc=False),
        out_type=jax.ShapeDtypeStruct((N_EDGES, 32), F32),
        scratch_types=[pltpu.VMEM((GC,), jnp.int32),
                       pltpu.VMEM((GC,), jnp.int32),
                       pltpu.VMEM((GC, 16), F32),
                       pltpu.VMEM((GC, 16), F32),
                       pltpu.VMEM_SHARED((N_NODES, 16), F32),
                       pltpu.SemaphoreType.DMA,
                       pltpu.SemaphoreType.DMA],
    )
    def _sc_gather_kernel(h_hbm, src_hbm, dst_hbm, x32_hbm,
                          sv, dv, rj, ri, hsh, sem1, sem2):
        sid = lax.axis_index("s")
        wid = sid * NC + lax.axis_index("c")
        base = wid * PER_W

        @pl.when(sid == 0)
        def _():
            pltpu.sync_copy(h_hbm, hsh)

        plsc.subcore_barrier()

        def body(j, carry):
            off = base + j * GC
            pltpu.sync_copy(src_hbm.at[pl.ds(off, GC)], sv)
            pltpu.sync_copy(dst_hbm.at[pl.ds(off, GC)], dv)
            cj = pltpu.async_copy(hsh.at[sv], rj, sem1)
            ci = pltpu.async_copy(hsh.at[dv], ri, sem2)
            cj.wait()
            ci.wait()
            pltpu.sync_copy(ri, x32_hbm.at[pl.ds(off, GC), pl.ds(0, 16)])
            pltpu.sync_copy(rj, x32_hbm.at[pl.ds(off, GC), pl.ds(16, 16)])
            return carry

        lax.fori_loop(0, ITERS, body, 0)

    return _sc_gather_kernel


@functools.cache
def _build_sc_scatter():
    mesh = plsc.VectorSubcoreMesh(core_axis_name="c", subcore_axis_name="s")

    @functools.partial(
        pl.kernel, mesh=mesh,
        compiler_params=pltpu.CompilerParams(use_tc_tiling_on_sc=False),
        out_type=jax.ShapeDtypeStruct((NC, N_NODES, 16), F32),
        scratch_types=[pltpu.VMEM((GS, 16), F32),
                       pltpu.VMEM((GS,), jnp.int32),
                       pltpu.VMEM_SHARED((N_NODES, 16), F32)],
    )
    def _sc_scatter_kernel(m_hbm, dst_hbm, zeros_hbm, out_hbm, mv, dv, acc):
        cid = lax.axis_index("c")
        sid = lax.axis_index("s")
        wid = sid * NC + cid

        @pl.when(sid == 0)
        def _():
            pltpu.sync_copy(zeros_hbm, acc)

        plsc.subcore_barrier()
        base = wid * PER_W

        def body(j, carry):
            off = base + j * GS
            pltpu.sync_copy(m_hbm.at[pl.ds(off, GS)], mv)
            pltpu.sync_copy(dst_hbm.at[pl.ds(off, GS)], dv)
            pltpu.sync_copy(mv, acc.at[dv], add=True)
            return carry

        lax.fori_loop(0, ITERS_S, body, 0)
        plsc.subcore_barrier()
        rows = N_NODES // 16
        pltpu.sync_copy(acc.at[pl.ds(sid * rows, rows)],
                        out_hbm.at[cid, pl.ds(sid * rows, rows)])

    return _sc_scatter_kernel


@functools.cache
def _build_sc_counts():
    mesh = plsc.VectorSubcoreMesh(core_axis_name="c", subcore_axis_name="s")

    @functools.partial(
        pl.kernel, mesh=mesh,
        compiler_params=pltpu.CompilerParams(use_tc_tiling_on_sc=False),
        out_type=jax.ShapeDtypeStruct((NC, N_NODES, 16), F32),
        scratch_types=[pltpu.VMEM((GS, 16), F32),
                       pltpu.VMEM((GS,), jnp.int32),
                       pltpu.VMEM_SHARED((N_NODES, 16), F32)],
    )
    def _sc_counts_kernel(dst_hbm, ones_hbm, zeros_hbm, out_hbm, ov, dv, acc):
        cid = lax.axis_index("c")
        sid = lax.axis_index("s")
        wid = sid * NC + cid

        pltpu.sync_copy(ones_hbm, ov)

        @pl.when(sid == 0)
        def _():
            pltpu.sync_copy(zeros_hbm, acc)

        plsc.subcore_barrier()
        base = wid * PER_W

        def body(j, carry):
            off = base + j * GS
            pltpu.sync_copy(dst_hbm.at[pl.ds(off, GS)], dv)
            pltpu.sync_copy(ov, acc.at[dv], add=True)
            return carry

        lax.fori_loop(0, ITERS_S, body, 0)
        plsc.subcore_barrier()
        rows = N_NODES // 16
        pltpu.sync_copy(acc.at[pl.ds(sid * rows, rows)],
                        out_hbm.at[cid, pl.ds(sid * rows, rows)])

    return _sc_counts_kernel


def _sc_gather(h, src, dst):
    return _build_sc_gather()(h, src, dst)


def _sc_scatter(m, dst, zeros_n16):
    return _build_sc_scatter()(m, dst, zeros_n16)


def _sc_counts(dst, ones_gc16, zeros_n16):
    return _build_sc_counts()(dst, ones_gc16, zeros_n16)


def _dot_f32(a, b):
    return jax.lax.dot_general(a, b, (((1,), (0,)), ((), ())),
                               preferred_element_type=F32)


def _enc_body(x_ref, w_ref, b_ref, o_ref):
    o_ref[...] = x_ref[...] @ w_ref[...] + b_ref[...]


def _msg_body(x32_ref, ea_ref, w0ab, w0c, b0, w1, b1, w2, b2, w3, b3, o_ref):
    ea = ea_ref[...]
    wc = w0c[...]
    mea = (ea[:, 0:1] * wc[0:1, :] + ea[:, 1:2] * wc[1:2, :]
           + ea[:, 2:3] * wc[2:3, :] + b0[...])
    m = _dot_f32(x32_ref[...].astype(BF), w0ab[...]) + mea
    m = jnp.maximum(m, 0.0)
    m = jnp.maximum(_dot_f32(m.astype(BF), w1[...]) + b1[...], 0.0)
    m = jnp.maximum(_dot_f32(m.astype(BF), w2[...]) + b2[...], 0.0)
    o_ref[...] = _dot_f32(m.astype(BF), w3[...]) + b3[...]


def _upd_body(h_ref, s0_ref, s1_ref, c0_ref, c1_ref, w0a, w0b, b0, w1, b1, o_ref):
    cnt = jnp.maximum(c0_ref[...] + c1_ref[...], 1.0)
    aggr = (s0_ref[...] + s1_ref[...]) / cnt
    u = h_ref[...] @ w0a[...] + aggr @ w0b[...] + b0[...]
    u = jnp.maximum(u, 0.0)
    o_ref[...] = u @ w1[...] + b1[...]


def _final_body(h_ref, w0, b0, w1, b1, w2, b2, o_ref):
    p = jnp.maximum(h_ref[...] @ w0[...] + b0[...], 0.0)
    p = jnp.maximum(p @ w1[...] + b1[...], 0.0)
    p = p @ w2[...] + b2[...]
    part = jnp.sum(p, axis=0, keepdims=True)

    @pl.when(pl.program_id(0) == 0)
    def _():
        o_ref[...] = jnp.zeros_like(o_ref)

    o_ref[...] += part


def _full_spec(shape):
    return pl.BlockSpec(shape, lambda i: tuple(0 for _ in shape))


def _encoder(x, enc_Wt, enc_b2):
    return pl.pallas_call(
        _enc_body,
        grid=(N_NODES // BN,),
        in_specs=[
            pl.BlockSpec((BN, 16), lambda i: (i, 0)),
            _full_spec((16, 16)),
            _full_spec((1, 16)),
        ],
        out_specs=pl.BlockSpec((BN, 16), lambda i: (i, 0)),
        out_shape=jax.ShapeDtypeStruct((N_NODES, 16), F32),
    )(x, enc_Wt, enc_b2)


def _msg_mlp(x32, ea, w0ab, w0c, b0, w1, b1, w2, b2, w3, b3):
    return pl.pallas_call(
        _msg_body,
        grid=(N_EDGES // BE,),
        in_specs=[
            pl.BlockSpec((BE, 32), lambda i: (i, 0)),
            pl.BlockSpec((BE, 3), lambda i: (i, 0)),
            _full_spec((32, 70)),
            _full_spec((3, 70)),
            _full_spec((1, 70)),
            _full_spec((70, 140)),
            _full_spec((1, 140)),
            _full_spec((140, 20)),
            _full_spec((1, 20)),
            _full_spec((20, 16)),
            _full_spec((1, 16)),
        ],
        out_specs=pl.BlockSpec((BE, 16), lambda i: (i, 0)),
        out_shape=jax.ShapeDtypeStruct((N_EDGES, 16), F32),
    )(x32, ea, w0ab, w0c, b0, w1, b1, w2, b2, w3, b3)


def _update(h, s0, s1, c0, c1, w0a, w0b, b0, w1, b1):
    return pl.pallas_call(
        _upd_body,
        grid=(N_NODES // BN,),
        in_specs=[
            pl.BlockSpec((BN, 16), lambda i: (i, 0)),
            pl.BlockSpec((BN, 16), lambda i: (i, 0)),
            pl.BlockSpec((BN, 16), lambda i: (i, 0)),
            pl.BlockSpec((BN, 16), lambda i: (i, 0)),
            pl.BlockSpec((BN, 16), lambda i: (i, 0)),
            _full_spec((16, 70)),
            _full_spec((16, 70)),
            _full_spec((1, 70)),
            _full_spec((70, 16)),
            _full_spec((1, 16)),
        ],
        out_specs=pl.BlockSpec((BN, 16), lambda i: (i, 0)),
        out_shape=jax.ShapeDtypeStruct((N_NODES, 16), F32),
    )(h, s0, s1, c0, c1, w0a, w0b, b0, w1, b1)


def _final(h, w0, b0, w1, b1, w2, b2):
    out = pl.pallas_call(
        _final_body,
        grid=(N_NODES // BN,),
        in_specs=[
            pl.BlockSpec((BN, 16), lambda i: (i, 0)),
            _full_spec((16, 64)),
            _full_spec((1, 64)),
            _full_spec((64, 32)),
            _full_spec((1, 32)),
            _full_spec((32, 3)),
            _full_spec((1, 3)),
        ],
        out_specs=_full_spec((1, 3)),
        out_shape=jax.ShapeDtypeStruct((1, 3), F32),
    )(h, w0, b0, w1, b1, w2, b2)
    return out / N_NODES


def kernel(x, edge_index, edge_attr, enc_W, enc_b,
           mW0, mb0, mW1, mb1, mW2, mb2, mW3, mb3,
           uW0, ub0, uW1, ub1,
           fW0, fb0, fW1, fb1, fW2, fb2):
    src = edge_index[0]
    dst = edge_index[1]
    zeros_n16 = jnp.zeros((N_NODES, 16), F32)
    ones_gc16 = jnp.ones((GS, 16), F32)

    h = _encoder(x, enc_W.T, enc_b.reshape(1, 16))
    cnt = _sc_counts(dst, ones_gc16, zeros_n16)

    for l in range(3):
        x32 = _sc_gather(h, src, dst)
        w0t = mW0[l].T
        m = _msg_mlp(x32, edge_attr,
                     w0t[0:32].astype(BF), w0t[32:35],
                     mb0[l].reshape(1, 70),
                     mW1[l].T.astype(BF), mb1[l].reshape(1, 140),
                     mW2[l].T.astype(BF), mb2[l].reshape(1, 20),
                     mW3[l].T.astype(BF), mb3[l].reshape(1, 16))
        s = _sc_scatter(m, dst, zeros_n16)
        u0t = uW0[l].T
        h = _update(h, s[0], s[1], cnt[0], cnt[1],
                    u0t[0:16], u0t[16:32], ub0[l].reshape(1, 70),
                    uW1[l].T, ub1[l].reshape(1, 16))

    return _final(h, fW0.T, fb0.reshape(1, 64), fW1.T, fb1.reshape(1, 32),
                  fW2.T, fb2.reshape(1, 3))
